# baseline (speedup 1.0000x reference)
import functools

import jax
import jax.numpy as jnp
from jax import lax
from jax.experimental import pallas as pl
from jax.experimental.pallas import tpu as pltpu
from jax.experimental.pallas import tpu_sc as plsc

EMBD = 64
TAGS = 20
PTAGS = 32
QS = 4
NC, NS = 2, 16
NW = NC * NS
CHUNK = 128
NBUF = 5
VR = 8192


def _project_body(t_ref, m_ref, out_ref, *, vocab):
    slab = t_ref[...]
    width = slab.shape[1]
    col0 = pl.program_id(0) * width
    col = col0 + lax.broadcasted_iota(jnp.int32, slab.shape, 1)
    slab = jnp.where(col < vocab, slab, 0.0)
    stacked = jnp.concatenate(
        [slab[:, q * VR:(q + 1) * VR] for q in range(QS)], axis=0)
    out_ref[...] = lax.dot_general(
        stacked, m_ref[...], (((0,), (0,)), ((), ())),
        preferred_element_type=jnp.float32)


def _project(table_t, mcat, grid_n, interpret=False):
    vocab = table_t.shape[1]
    return pl.pallas_call(
        functools.partial(_project_body, vocab=vocab),
        grid=(grid_n,),
        in_specs=[
            pl.BlockSpec((EMBD, QS * VR), lambda i: (0, i)),
            pl.BlockSpec((QS * EMBD, QS * PTAGS), lambda i: (0, 0)),
        ],
        out_specs=pl.BlockSpec((VR, QS * PTAGS), lambda i: (i, 0)),
        out_shape=jax.ShapeDtypeStruct((grid_n * VR, QS * PTAGS),
                                       jnp.float32),
        compiler_params=pltpu.CompilerParams(
            dimension_semantics=("parallel",)),
        interpret=interpret,
    )(table_t, mcat)


def _sc_gather_body(idx_hbm, p_hbm, out_hbm, idx_v, *rest):
    bufs = rest[:NBUF]
    gsems = rest[NBUF:2 * NBUF]
    osems = rest[2 * NBUF:3 * NBUF]
    cpw = idx_v.shape[0]
    groups = cpw // NBUF
    wid = lax.axis_index("s") * NC + lax.axis_index("c")
    chunk0 = wid * cpw

    pltpu.sync_copy(idx_hbm.at[wid], idx_v)

    def starts(j, c):
        pltpu.async_copy(p_hbm.at[idx_v.at[c]], bufs[j], gsems[j])

    def waits(j):
        pltpu.make_async_copy(p_hbm.at[idx_v.at[0]], bufs[j],
                              gsems[j]).wait()

    def out_starts(j, c):
        base = (chunk0 + c) * CHUNK
        pltpu.async_copy(bufs[j], out_hbm.at[pl.ds(base, CHUNK)], osems[j])

    def out_waits(j):
        pltpu.make_async_copy(bufs[j], out_hbm.at[pl.ds(0, CHUNK)],
                              osems[j]).wait()

    for j in range(NBUF):
        starts(j, j)

    @pl.loop(0, groups - 1)
    def _(k):
        for j in range(NBUF):
            c = k * NBUF + j
            waits(j)
            out_starts(j, c)
            out_waits(j)
            starts(j, c + NBUF)

    for j in range(NBUF):
        c = cpw - NBUF + j
        waits(j)
        out_starts(j, c)
        out_waits(j)


def _sc_gather(idx3d, p_flat, n_tokens):
    mesh = plsc.VectorSubcoreMesh(core_axis_name="c", subcore_axis_name="s")
    cpw = idx3d.shape[1]
    run = pl.kernel(
        _sc_gather_body,
        out_type=jax.ShapeDtypeStruct((n_tokens, PTAGS), jnp.float32),
        mesh=mesh,
        scratch_types=(
            [pltpu.VMEM((cpw, CHUNK), jnp.int32)]
            + [pltpu.VMEM((CHUNK, PTAGS), jnp.float32) for _ in range(NBUF)]
            + [pltpu.SemaphoreType.DMA for _ in range(2 * NBUF)]
        ),
        compiler_params=pltpu.CompilerParams(use_tc_tiling_on_sc=False),
    )
    return run(idx3d, p_flat)


def _final_body(pl_ref, b_ref, ml0, ml1, ml2, ml3,
                pp_ref, loss_ref, acc_ref):
    i = pl.program_id(0)

    @pl.when(i == 0)
    def _():
        acc_ref[0] = 0.0
        acc_ref[1] = 0.0

    r = pl_ref.shape[0] // (QS * PTAGS)
    slab = pl_ref[...].reshape(r, QS * PTAGS)
    slab_t = jnp.transpose(slab, (1, 0))
    iota0 = lax.broadcasted_iota(jnp.int32, (TAGS, r), 0)
    bias = b_ref[...][:, None]
    mls = (ml0, ml1, ml2, ml3)
    nll_tot = 0.0
    cnt_tot = 0.0
    packed = jnp.zeros((r,), jnp.int32)
    for q in range(QS):
        out = slab_t[q * PTAGS:q * PTAGS + TAGS, :] + bias
        m = jnp.max(out, axis=0, keepdims=True)
        pred = jnp.min(jnp.where(out == m, iota0, TAGS), axis=0)
        packed = packed | (pred << (5 * q))
        lse = m[0] + jnp.log(jnp.sum(jnp.exp(out - m), axis=0))
        ml = mls[q][...]
        msk = ml >> 5
        lab = ml & 31
        labm = jnp.where(msk == 0, -1, lab)
        valid = labm != -1
        safe = jnp.where(valid, labm, 0)
        picked = jnp.sum(jnp.where(iota0 == safe[None, :], out, 0.0), axis=0)
        nll_tot += jnp.sum(jnp.where(valid, lse - picked, 0.0))
        cnt_tot += jnp.sum(valid.astype(jnp.float32))
    pp_ref[...] = packed
    acc_ref[0] += nll_tot
    acc_ref[1] += cnt_tot

    @pl.when(i == pl.num_programs(0) - 1)
    def _():
        loss_ref[0, 0] = acc_ref[0] / jnp.maximum(acc_ref[1], 1.0)


def _finalize(plog1d, b, ml_strips, rows_per_step=2048, interpret=False):
    t4 = plog1d.shape[0] // (QS * PTAGS)
    grid = (t4 // rows_per_step,)
    r = rows_per_step
    row_spec = pl.BlockSpec((r,), lambda i: (i,))
    outs = pl.pallas_call(
        _final_body,
        grid=grid,
        in_specs=[
            pl.BlockSpec((r * QS * PTAGS,), lambda i: (i,)),
            pl.BlockSpec((TAGS,), lambda i: (0,)),
        ] + [row_spec] * 4,
        out_specs=[row_spec, pl.BlockSpec(memory_space=pltpu.SMEM)],
        out_shape=[jax.ShapeDtypeStruct((t4,), jnp.int32),
                   jax.ShapeDtypeStruct((1, 1), jnp.float32)],
        scratch_shapes=[pltpu.SMEM((2,), jnp.float32)],
        compiler_params=pltpu.CompilerParams(
            dimension_semantics=("arbitrary",)),
        interpret=interpret,
    )(plog1d, b, *ml_strips)
    return outs[0], outs[1]


def kernel(token_ids, mask, labels, table, W, b):
    bsz, seq = token_ids.shape
    t = bsz * seq
    t4 = t // QS
    vocab, embd = table.shape
    slab = QS * VR
    grid_n = -(-vocab // slab)

    table_t = table.T

    wt = W.T
    wpad = jnp.pad(wt, ((0, 0), (0, PTAGS - TAGS)))
    eye = jnp.eye(QS, dtype=wpad.dtype)
    mcat = jnp.einsum("ec,qp->qepc", wpad, eye).reshape(
        QS * EMBD, QS * PTAGS)

    p = _project(table_t, mcat, grid_n)
    p_flat = p.reshape(grid_n * slab, PTAGS)

    i = token_ids // slab
    rem = token_ids - i * slab
    q = rem // VR
    rr = rem - q * VR
    idx = (i * VR + rr) * QS + q

    idx_sm = idx.reshape(QS, t4).T
    idx3d = idx_sm.reshape(NW, t // (NW * CHUNK), CHUNK)

    plog = _sc_gather(idx3d, p_flat, t)
    plog1d = plog.reshape(t * PTAGS)

    ml_flat = (mask * 32 + labels).reshape(t).astype(jnp.int32)
    ml_strips = tuple(lax.slice(ml_flat, (qq * t4,), ((qq + 1) * t4,))
                      for qq in range(QS))

    pp, loss = _finalize(plog1d, b, ml_strips)
    pred = (pp[None, :] >> (5 * jnp.arange(QS, dtype=jnp.int32))[:, None]
            ) & 31
    return pred.reshape(bsz, seq), loss[0, 0]


# speedup vs baseline: 1.1739x; 1.1739x over previous
import functools

import jax
import jax.numpy as jnp
from jax import lax
from jax.experimental import pallas as pl
from jax.experimental.pallas import tpu as pltpu
from jax.experimental.pallas import tpu_sc as plsc

EMBD = 64
TAGS = 20
PTAGS = 32
QS = 4
NC, NS = 2, 16
NW = NC * NS
CHUNK = 128
NBUF = 5
VR = 8192


def _project_body(t_ref, m_ref, out_ref, *, vocab):
    slab = t_ref[...]
    width = slab.shape[1]
    col0 = pl.program_id(0) * width
    col = col0 + lax.broadcasted_iota(jnp.int32, slab.shape, 1)
    slab = jnp.where(col < vocab, slab, 0.0)
    stacked = jnp.concatenate(
        [slab[:, q * VR:(q + 1) * VR] for q in range(QS)], axis=0)
    out_ref[...] = lax.dot_general(
        stacked, m_ref[...], (((0,), (0,)), ((), ())),
        preferred_element_type=jnp.float32)


def _project(table_t, mcat, grid_n, interpret=False):
    vocab = table_t.shape[1]
    return pl.pallas_call(
        functools.partial(_project_body, vocab=vocab),
        grid=(grid_n,),
        in_specs=[
            pl.BlockSpec((EMBD, QS * VR), lambda i: (0, i)),
            pl.BlockSpec((QS * EMBD, QS * PTAGS), lambda i: (0, 0)),
        ],
        out_specs=pl.BlockSpec((VR, QS * PTAGS), lambda i: (i, 0)),
        out_shape=jax.ShapeDtypeStruct((grid_n * VR, QS * PTAGS),
                                       jnp.float32),
        compiler_params=pltpu.CompilerParams(
            dimension_semantics=("parallel",)),
        interpret=interpret,
    )(table_t, mcat)


def _sc_gather_body(idx_hbm, p_hbm, out_hbm, idx_v, *rest):
    bufs = rest[:NBUF]
    gsems = rest[NBUF:2 * NBUF]
    osems = rest[2 * NBUF:3 * NBUF]
    cpw = idx_v.shape[0]
    groups = cpw // NBUF
    wid = lax.axis_index("s") * NC + lax.axis_index("c")
    chunk0 = wid * cpw

    pltpu.sync_copy(idx_hbm.at[wid], idx_v)

    def starts(j, c):
        pltpu.async_copy(p_hbm.at[idx_v.at[c]], bufs[j], gsems[j])

    def waits(j):
        pltpu.make_async_copy(p_hbm.at[idx_v.at[0]], bufs[j],
                              gsems[j]).wait()

    cps = out_hbm.shape[0] // CHUNK

    def out_starts(j, c):
        cc = chunk0 + c
        qq = cc // cps
        k0 = (cc - qq * cps) * CHUNK
        pltpu.async_copy(bufs[j], out_hbm.at[pl.ds(k0, CHUNK), qq],
                         osems[j])

    def out_waits(j):
        pltpu.make_async_copy(bufs[j], out_hbm.at[pl.ds(0, CHUNK), 0],
                              osems[j]).wait()

    for j in range(NBUF):
        starts(j, j)

    @pl.loop(0, groups - 1)
    def _(k):
        for j in range(NBUF):
            c = k * NBUF + j
            waits(j)
            out_starts(j, c)
            out_waits(j)
            starts(j, c + NBUF)

    for j in range(NBUF):
        c = cpw - NBUF + j
        waits(j)
        out_starts(j, c)
        out_waits(j)


def _sc_gather(idx3d, p_flat, n_tokens):
    mesh = plsc.VectorSubcoreMesh(core_axis_name="c", subcore_axis_name="s")
    cpw = idx3d.shape[1]
    run = pl.kernel(
        _sc_gather_body,
        out_type=jax.ShapeDtypeStruct((n_tokens // QS, QS, PTAGS),
                                      jnp.float32),
        mesh=mesh,
        scratch_types=(
            [pltpu.VMEM((cpw, CHUNK), jnp.int32)]
            + [pltpu.VMEM((CHUNK, PTAGS), jnp.float32) for _ in range(NBUF)]
            + [pltpu.SemaphoreType.DMA for _ in range(2 * NBUF)]
        ),
        compiler_params=pltpu.CompilerParams(use_tc_tiling_on_sc=False),
    )
    return run(idx3d, p_flat)


def _final_body(pl_ref, b_ref, ml0, ml1, ml2, ml3,
                pp_ref, loss_ref, acc_ref):
    i = pl.program_id(0)

    @pl.when(i == 0)
    def _():
        acc_ref[0] = 0.0
        acc_ref[1] = 0.0

    r = pl_ref.shape[0] // (QS * PTAGS)
    slab = pl_ref[...].reshape(r, QS * PTAGS)
    slab_t = jnp.transpose(slab, (1, 0))
    iota0 = lax.broadcasted_iota(jnp.int32, (TAGS, r), 0)
    bias = b_ref[...][:, None]
    mls = (ml0, ml1, ml2, ml3)
    nll_tot = 0.0
    cnt_tot = 0.0
    packed = jnp.zeros((r,), jnp.int32)
    for q in range(QS):
        out = slab_t[q * PTAGS:q * PTAGS + TAGS, :] + bias
        m = jnp.max(out, axis=0, keepdims=True)
        pred = jnp.min(jnp.where(out == m, iota0, TAGS), axis=0)
        packed = packed | (pred << (5 * q))
        lse = m[0] + jnp.log(jnp.sum(jnp.exp(out - m), axis=0))
        ml = mls[q][...]
        msk = ml >> 5
        lab = ml & 31
        labm = jnp.where(msk == 0, -1, lab)
        valid = labm != -1
        safe = jnp.where(valid, labm, 0)
        picked = jnp.sum(jnp.where(iota0 == safe[None, :], out, 0.0), axis=0)
        nll_tot += jnp.sum(jnp.where(valid, lse - picked, 0.0))
        cnt_tot += jnp.sum(valid.astype(jnp.float32))
    pp_ref[...] = packed
    acc_ref[0] += nll_tot
    acc_ref[1] += cnt_tot

    @pl.when(i == pl.num_programs(0) - 1)
    def _():
        loss_ref[0, 0] = acc_ref[0] / jnp.maximum(acc_ref[1], 1.0)


def _finalize(plog1d, b, ml_strips, rows_per_step=2048, interpret=False):
    t4 = plog1d.shape[0] // (QS * PTAGS)
    grid = (t4 // rows_per_step,)
    r = rows_per_step
    row_spec = pl.BlockSpec((r,), lambda i: (i,))
    outs = pl.pallas_call(
        _final_body,
        grid=grid,
        in_specs=[
            pl.BlockSpec((r * QS * PTAGS,), lambda i: (i,)),
            pl.BlockSpec((TAGS,), lambda i: (0,)),
        ] + [row_spec] * 4,
        out_specs=[row_spec, pl.BlockSpec(memory_space=pltpu.SMEM)],
        out_shape=[jax.ShapeDtypeStruct((t4,), jnp.int32),
                   jax.ShapeDtypeStruct((1, 1), jnp.float32)],
        scratch_shapes=[pltpu.SMEM((2,), jnp.float32)],
        compiler_params=pltpu.CompilerParams(
            dimension_semantics=("arbitrary",)),
        interpret=interpret,
    )(plog1d, b, *ml_strips)
    return outs[0], outs[1]


def kernel(token_ids, mask, labels, table, W, b):
    bsz, seq = token_ids.shape
    t = bsz * seq
    t4 = t // QS
    vocab, embd = table.shape
    slab = QS * VR
    grid_n = -(-vocab // slab)

    table_t = table.T

    wt = W.T
    wpad = jnp.pad(wt, ((0, 0), (0, PTAGS - TAGS)))
    eye = jnp.eye(QS, dtype=wpad.dtype)
    mcat = jnp.einsum("ec,qp->qepc", wpad, eye).reshape(
        QS * EMBD, QS * PTAGS)

    p = _project(table_t, mcat, grid_n)
    p_flat = p.reshape(grid_n * slab, PTAGS)

    i = token_ids // slab
    rem = token_ids - i * slab
    q = rem // VR
    rr = rem - q * VR
    idx = (i * VR + rr) * QS + q
    idx3d = idx.reshape(NW, t // (NW * CHUNK), CHUNK)

    plog = _sc_gather(idx3d, p_flat, t)
    plog1d = plog.reshape(t * PTAGS)

    ml_flat = (mask * 32 + labels).reshape(t).astype(jnp.int32)
    ml_strips = tuple(lax.slice(ml_flat, (qq * t4,), ((qq + 1) * t4,))
                      for qq in range(QS))

    pp, loss = _finalize(plog1d, b, ml_strips)
    pred = (pp[None, :] >> (5 * jnp.arange(QS, dtype=jnp.int32))[:, None]
            ) & 31
    return pred.reshape(bsz, seq), loss[0, 0]
